# baseline (speedup 1.0000x reference)
import jax
import jax.numpy as jnp
from jax import lax
from jax.experimental import pallas as pl
from jax.experimental.pallas import tpu as pltpu

_B = 64
_S = 2048
_H = 16
_KVH = 4
_G = _H // _KVH
_DH = 128
_HALF = _DH // 2
_HID = 2048
_THETA = 10000.0
_CTX = 4096
_SCALE = _DH ** -0.5
_SK = _S * _KVH


def _matmul_body(x_ref, w_ref, o_ref):
    o_ref[:, :] = jnp.dot(x_ref[:, :], w_ref[:, :],
                          preferred_element_type=jnp.float32)


def _matmul(x, w, nblk):
    m, k = x.shape
    n = w.shape[1]
    blk = n // nblk
    return pl.pallas_call(
        _matmul_body,
        grid=(nblk,),
        in_specs=[
            pl.BlockSpec((m, k), lambda j: (0, 0)),
            pl.BlockSpec((k, blk), lambda j: (0, j)),
        ],
        out_specs=pl.BlockSpec((m, blk), lambda j: (0, j)),
        out_shape=jax.ShapeDtypeStruct((m, n), jnp.float32),
    )(x, w)


def _attn_body(qa_ref, kn_ref, vn_ref, cq_ref, sq_ref, c2_ref, s2_ref,
               k_ref, v_ref, o_ref):
    cq = cq_ref[0]
    sq = sq_ref[0]

    q = qa_ref[0]
    q1 = q[:, :_HALF]
    q2 = q[:, _HALF:]
    qr1 = (q1 * cq - q2 * sq) * _SCALE
    qr2 = (q2 * cq + q1 * sq) * _SCALE
    qa = jnp.concatenate([qr1, qr2], axis=1)
    qb = jnp.concatenate([qr2, -qr1], axis=1)

    kn = kn_ref[0]
    kn1 = kn[:, :_HALF]
    kn2 = kn[:, _HALF:]
    knr = jnp.concatenate([kn1 * cq - kn2 * sq, kn2 * cq + kn1 * sq], axis=1)

    kc = k_ref[0]
    a = kc * c2_ref[:, :]
    bm = kc * s2_ref[:, :]
    scores = (lax.dot_general(qa, a, (((1,), (1,)), ((), ()))) +
              lax.dot_general(qb, bm, (((1,), (1,)), ((), ()))))

    r_kvh = lax.broadcasted_iota(jnp.int32, (_H, _SK), 1) % _KVH
    h_kvh = lax.broadcasted_iota(jnp.int32, (_H, _SK), 0) // _G
    valid = r_kvh == h_kvh

    s_new = lax.dot_general(qa, knr, (((1,), (1,)), ((), ())))
    diag = (lax.broadcasted_iota(jnp.int32, (_H, _H), 0) ==
            lax.broadcasted_iota(jnp.int32, (_H, _H), 1))

    masked = jnp.where(valid, scores, -1e30)
    masked_new = jnp.where(diag, s_new, -1e30)
    m = jnp.maximum(jnp.max(masked, axis=1, keepdims=True),
                    jnp.max(masked_new, axis=1, keepdims=True))
    e = jnp.where(valid, jnp.exp(scores - m), 0.0)
    e_new = jnp.where(diag, jnp.exp(s_new - m), 0.0)
    denom = (jnp.sum(e, axis=1, keepdims=True) +
             jnp.sum(e_new, axis=1, keepdims=True))

    vc = v_ref[0]
    acc = (lax.dot_general(e, vc, (((1,), (0,)), ((), ()))) +
           lax.dot_general(e_new, vn_ref[0], (((1,), (0,)), ((), ()))))
    o_ref[0] = acc / denom


def _attention(qa, kn, vn, cq, sq, c2, s2, kc, vc):
    return pl.pallas_call(
        _attn_body,
        grid=(_B,),
        in_specs=[
            pl.BlockSpec((1, _H, _DH), lambda b: (b, 0, 0)),
            pl.BlockSpec((1, _H, _DH), lambda b: (b, 0, 0)),
            pl.BlockSpec((1, _H, _DH), lambda b: (b, 0, 0)),
            pl.BlockSpec((1, 1, _HALF), lambda b: (b, 0, 0)),
            pl.BlockSpec((1, 1, _HALF), lambda b: (b, 0, 0)),
            pl.BlockSpec((_SK, _DH), lambda b: (0, 0)),
            pl.BlockSpec((_SK, _DH), lambda b: (0, 0)),
            pl.BlockSpec((1, _SK, _DH), lambda b: (b, 0, 0)),
            pl.BlockSpec((1, _SK, _DH), lambda b: (b, 0, 0)),
        ],
        out_specs=pl.BlockSpec((1, _H, _DH), lambda b: (b, 0, 0)),
        out_shape=jax.ShapeDtypeStruct((_B, _H, _DH), jnp.float32),
        compiler_params=pltpu.CompilerParams(
            dimension_semantics=("arbitrary",)),
    )(qa, kn, vn, cq, sq, c2, s2, kc, vc)


def kernel(positions, hidden_states, k_cache, v_cache, Wqkv, Wo):
    qkv = _matmul(hidden_states, Wqkv, 6)

    qa = qkv[:, :_H * _DH].reshape(_B, _H, _DH)
    kn = qkv[:, _H * _DH:(_H + _KVH) * _DH].reshape(_B, _KVH, _DH)
    vn = qkv[:, (_H + _KVH) * _DH:].reshape(_B, _KVH, _DH)
    kn = jnp.repeat(kn, _G, axis=1)
    vn = jnp.repeat(vn, _G, axis=1)

    inv_freq = 1.0 / (_THETA ** (jnp.arange(0, _DH, 2, dtype=jnp.float32)
                                 / _DH))
    pos = jnp.minimum(positions, _CTX - 1).astype(jnp.float32)
    fq = pos[:, None] * inv_freq[None, :]
    cq = jnp.cos(fq)[:, None, :]
    sq = jnp.sin(fq)[:, None, :]
    past = jnp.minimum(jnp.arange(_S, dtype=jnp.int32),
                       _CTX - 1).astype(jnp.float32)
    fp = past[:, None] * inv_freq[None, :]
    c2 = jnp.repeat(jnp.tile(jnp.cos(fp), (1, 2)), _KVH, axis=0)
    s2 = jnp.repeat(jnp.tile(jnp.sin(fp), (1, 2)), _KVH, axis=0)

    kc = k_cache.reshape(_B, _SK, _DH)
    vc = v_cache.reshape(_B, _SK, _DH)

    attn = _attention(qa, kn, vn, cq, sq, c2, s2, kc, vc)
    attn = attn.reshape(_B, _H * _DH)

    return _matmul(attn, Wo, 4)
